# baseline (speedup 1.0000x reference)
import functools
import jax
import jax.numpy as jnp
from jax.experimental import pallas as pl

N = 10000
E = 160000
F_IN = 128
H = 32
G = 312

NB = 1000
EB = 640


def _dot(a, b):
    return jax.lax.dot_general(a, b, (((1,), (0,)), ((), ())),
                               preferred_element_type=jnp.float32,
                               precision=jax.lax.Precision.HIGHEST)


def _elu(x):
    return jnp.where(x > 0, x, jnp.exp(jnp.minimum(x, 0.0)) - 1.0)


def _tc1_body(x_ref, w0_ref, b0_ref, mw1_ref, mb1_ref, h_ref, t_ref):
    h = _dot(x_ref[...], w0_ref[...]) + b0_ref[...]
    h_ref[...] = h
    t_ref[...] = jnp.maximum(_dot(h, mw1_ref[...]) + mb1_ref[...], 0.0)


def _tc1(x, w0, b0, mw1, mb1):
    grid = (N // NB,)
    full = lambda a: pl.BlockSpec(a.shape, lambda i: (0,) * a.ndim)
    return pl.pallas_call(
        _tc1_body,
        grid=grid,
        in_specs=[pl.BlockSpec((NB, F_IN), lambda i: (i, 0)),
                  full(w0), full(b0), full(mw1), full(mb1)],
        out_specs=[pl.BlockSpec((NB, H), lambda i: (i, 0)),
                   pl.BlockSpec((NB, H), lambda i: (i, 0))],
        out_shape=[jax.ShapeDtypeStruct((N, H), jnp.float32),
                   jax.ShapeDtypeStruct((N, H), jnp.float32)],
    )(x, w0, b0, mw1, mb1)


def _tc2_body(t_ref, agg_ref, h_ref, mw2_ref, mb2_ref, xm_ref, m_ref):
    m = jax.nn.sigmoid(_dot(t_ref[...] + agg_ref[...], mw2_ref[...]) + mb2_ref[...])
    m_ref[...] = m
    xm_ref[...] = h_ref[...] * m


def _tc2(t, agg, h, mw2, mb2):
    grid = (N // NB,)
    full = lambda a: pl.BlockSpec(a.shape, lambda i: (0,) * a.ndim)
    return pl.pallas_call(
        _tc2_body,
        grid=grid,
        in_specs=[pl.BlockSpec((NB, H), lambda i: (i, 0)),
                  pl.BlockSpec((NB, H), lambda i: (i, 0)),
                  pl.BlockSpec((NB, H), lambda i: (i, 0)),
                  full(mw2), full(mb2)],
        out_specs=[pl.BlockSpec((NB, H), lambda i: (i, 0)),
                   pl.BlockSpec((NB, 1), lambda i: (i, 0))],
        out_shape=[jax.ShapeDtypeStruct((N, H), jnp.float32),
                   jax.ShapeDtypeStruct((N, 1), jnp.float32)],
    )(t, agg, h, mw2, mb2)


def _tc3_body(ea_ref, xs_ref, a1_ref, c1_ref, a2p_ref, c2p_ref, sel_ref, msg_ref):
    u = jnp.maximum(_dot(ea_ref[...], a1_ref[...]) + c1_ref[...], 0.0)
    w = _dot(u, a2p_ref[...]) + c2p_ref[...]
    xs = xs_ref[...]
    xt = jnp.concatenate([xs] * H, axis=1)
    msg_ref[...] = _dot(w * xt, sel_ref[...])


def _tc3(ea, xs, a1, c1, a2p, c2p, sel):
    grid = (E // EB,)
    full = lambda a: pl.BlockSpec(a.shape, lambda i: (0,) * a.ndim)
    return pl.pallas_call(
        _tc3_body,
        grid=grid,
        in_specs=[pl.BlockSpec((EB, 5), lambda i: (i, 0)),
                  pl.BlockSpec((EB, H), lambda i: (i, 0)),
                  full(a1), full(c1), full(a2p), full(c2p), full(sel)],
        out_specs=pl.BlockSpec((EB, H), lambda i: (i, 0)),
        out_shape=jax.ShapeDtypeStruct((E, H), jnp.float32),
    )(ea, xs, a1, c1, a2p, c2p, sel)


def _tc4_body(xm_ref, agg_ref, wroot_ref, m_ref, mw1_ref, mb1_ref, h_ref, t_ref):
    h = _elu(_dot(xm_ref[...], wroot_ref[...]) + agg_ref[...])
    h_ref[...] = h
    hm = h * m_ref[...]
    t_ref[...] = jnp.maximum(_dot(hm, mw1_ref[...]) + mb1_ref[...], 0.0)


def _tc4(xm, agg, wroot, m, mw1, mb1):
    grid = (N // NB,)
    full = lambda a: pl.BlockSpec(a.shape, lambda i: (0,) * a.ndim)
    return pl.pallas_call(
        _tc4_body,
        grid=grid,
        in_specs=[pl.BlockSpec((NB, H), lambda i: (i, 0)),
                  pl.BlockSpec((NB, H), lambda i: (i, 0)),
                  full(wroot),
                  pl.BlockSpec((NB, 1), lambda i: (i, 0)),
                  full(mw1), full(mb1)],
        out_specs=[pl.BlockSpec((NB, H), lambda i: (i, 0)),
                   pl.BlockSpec((NB, H), lambda i: (i, 0))],
        out_shape=[jax.ShapeDtypeStruct((N, H), jnp.float32),
                   jax.ShapeDtypeStruct((N, H), jnp.float32)],
    )(xm, agg, wroot, m, mw1, mb1)


def _tc4f_body(xm_ref, agg_ref, wroot_ref, h_ref):
    h_ref[...] = _elu(_dot(xm_ref[...], wroot_ref[...]) + agg_ref[...])


def _tc4f(xm, agg, wroot):
    grid = (N // NB,)
    full = lambda a: pl.BlockSpec(a.shape, lambda i: (0,) * a.ndim)
    return pl.pallas_call(
        _tc4f_body,
        grid=grid,
        in_specs=[pl.BlockSpec((NB, H), lambda i: (i, 0)),
                  pl.BlockSpec((NB, H), lambda i: (i, 0)),
                  full(wroot)],
        out_specs=pl.BlockSpec((NB, H), lambda i: (i, 0)),
        out_shape=jax.ShapeDtypeStruct((N, H), jnp.float32),
    )(xm, agg, wroot)


def _tc5_body(p_ref, w1_ref, b1_ref, w2_ref, b2_ref, w3_ref, b3_ref, o_ref):
    o = _elu(_dot(p_ref[...], w1_ref[...]) + b1_ref[...])
    o = _elu(_dot(o, w2_ref[...]) + b2_ref[...])
    o_ref[...] = _dot(o, w3_ref[...]) + b3_ref[...]


def _tc5(pooled, w1, b1, w2, b2, w3, b3):
    full = lambda a: pl.BlockSpec(a.shape, lambda *_: (0,) * a.ndim)
    return pl.pallas_call(
        _tc5_body,
        in_specs=[full(pooled), full(w1), full(b1), full(w2), full(b2),
                  full(w3), full(b3)],
        out_specs=full(jnp.zeros((G, 1))),
        out_shape=jax.ShapeDtypeStruct((G, 1), jnp.float32),
    )(pooled, w1, b1, w2, b2, w3, b3)


def _segsum_gather(table, src, dst, nseg):
    return jax.ops.segment_sum(table[src], dst, num_segments=nseg)


def _gather(table, src):
    return table[src]


def _segsum(vals, dst, nseg):
    return jax.ops.segment_sum(vals, dst, num_segments=nseg)


def kernel(x, edge_index, edge_attr, batch, params):
    src = edge_index[0]
    dst = edge_index[1]
    p = params
    row = lambda v: v.reshape(1, -1)

    a2p = [p["A2"][i].reshape(F_IN, H, H).transpose(0, 2, 1).reshape(F_IN, H * H)
           for i in range(3)]
    c2p = [p["c2"][i].reshape(H, H).T.reshape(1, H * H) for i in range(3)]
    sel = (jnp.arange(H * H, dtype=jnp.int32)[:, None] // H
           == jnp.arange(H, dtype=jnp.int32)[None, :]).astype(jnp.float32)

    h, t = _tc1(x, p["W0"], row(p["b0"]), p["Mw1"][0], row(p["Mb1"][0]))
    for i in range(3):
        agg = _segsum_gather(t, src, dst, N)
        xm, m = _tc2(t, agg, h, p["Mw2"][i], row(p["Mb2"][i]))
        xs = _gather(xm, src)
        msg = _tc3(edge_attr, xs, p["A1"][i], row(p["c1"][i]), a2p[i], c2p[i], sel)
        agg2 = _segsum(msg, dst, N)
        if i < 2:
            h, t = _tc4(xm, agg2, p["Wroot"][i], m,
                        p["Mw1"][i + 1], row(p["Mb1"][i + 1]))
        else:
            h = _tc4f(xm, agg2, p["Wroot"][i])
    pooled = _segsum(h, batch, G)
    o = _tc5(pooled, p["W1"], row(p["b1"]), p["W2"], row(p["b2"]),
             p["W3"], row(p["b3"]))
    return o.reshape(-1)
